# baseline (speedup 1.0000x reference)
import functools

import numpy as np
import jax
import jax.numpy as jnp
from jax import lax
from jax.experimental import pallas as pl
from jax.experimental.pallas import tpu as pltpu
from jax.experimental.pallas import tpu_sc as plsc

N = 10000
E = 160000
FULL = 16
C = 64
CUTOFF = 12.0
RESCALE = 23.395238876342773
COLS = (0, 2, 6, 12)

B_EDGE = 1280
BATCH = 128
PER_TILE = E // 16
NB_FULL = PER_TILE // BATCH
TAIL = PER_TILE - NB_FULL * BATCH
ROWS_T = 624


def _emb_body(xe_ref, dist_ref, w4_ref, w1_ref, b1_ref, g_ref, bb_ref,
              w2_ref, b2_ref, s_ref, out_ref):
    xe = xe_ref[...]
    bsz = xe.shape[0]
    h = jnp.dot(xe, w1_ref[...], preferred_element_type=jnp.float32) + b1_ref[...]
    mu = jnp.mean(h, axis=-1, keepdims=True)
    var = jnp.mean((h - mu) ** 2, axis=-1, keepdims=True)
    h = (h - mu) * lax.rsqrt(var + 1e-5) * g_ref[...] + bb_ref[...]
    h = h * (1.0 / (1.0 + jnp.exp(-h)))
    rad = jnp.dot(h, w2_ref[...], preferred_element_type=jnp.float32) + b2_ref[...]
    d = dist_ref[...].reshape(1, bsz) * (1.0 / CUTOFF)
    d5 = (d * d) * (d * d) * d
    env = 1.0 + d5 * (-21.0 + d * (35.0 - 15.0 * d))
    env = jnp.where(d < 1.0, env, 0.0) * (1.0 / RESCALE)
    rad = rad * jnp.transpose(env, (1, 0))
    w4 = w4_ref[...]
    s_mat = s_ref[...]
    acc = None
    for l in range(4):
        wl = jnp.dot(w4[:, l * 16:(l + 1) * 16], s_mat,
                     preferred_element_type=jnp.float32)
        rl = rad[:, l * 64:(l + 1) * 64]
        r2 = jnp.tile(rl, (1, 16))
        term = wl * r2
        acc = term if acc is None else acc + term
    out_ref[...] = acc


def _compute_emb(x_edge, edge_distance, wigner_inv, W1, b1, ln_g, ln_b, W2, b2):
    grid = E // B_EDGE
    dist2 = edge_distance.reshape(grid, 1, B_EDGE)
    w4f = jnp.concatenate([wigner_inv[:, :, c] for c in COLS], axis=-1)
    s = np.zeros((16, 1024), dtype=np.float32)
    for f in range(1024):
        s[f // 64, f] = 1.0
    const = lambda shape: pl.BlockSpec(shape, lambda i: (0, 0))
    return pl.pallas_call(
        _emb_body,
        grid=(grid,),
        in_specs=[
            pl.BlockSpec((B_EDGE, 64), lambda i: (i, 0)),
            pl.BlockSpec((1, 1, B_EDGE), lambda i: (i, 0, 0)),
            pl.BlockSpec((B_EDGE, 64), lambda i: (i, 0)),
            const((64, 64)),
            const((1, 64)),
            const((1, 64)),
            const((1, 64)),
            const((64, 256)),
            const((1, 256)),
            const((16, 1024)),
        ],
        out_specs=pl.BlockSpec((B_EDGE, 1024), lambda i: (i, 0)),
        out_shape=jax.ShapeDtypeStruct((E, 1024), jnp.float32),
    )(x_edge, dist2, w4f, W1, b1.reshape(1, 64), ln_g.reshape(1, 64),
      ln_b.reshape(1, 64), W2, b2.reshape(1, 256), jnp.asarray(s))


DEPTH = 2


def _sc_scatter(emb2, dst, x2):
    mesh = plsc.VectorSubcoreMesh(core_axis_name="c", subcore_axis_name="s")

    @functools.partial(
        pl.kernel,
        mesh=mesh,
        out_type=jax.ShapeDtypeStruct((N, 1024), jnp.float32),
        scratch_types=[
            *[pltpu.VMEM((BATCH,), jnp.int32) for _ in range(DEPTH)],
            *[pltpu.VMEM((BATCH, 128), jnp.float32) for _ in range(DEPTH)],
            pltpu.VMEM((TAIL,), jnp.int32),
            pltpu.VMEM((TAIL, 128), jnp.float32),
            pltpu.VMEM_SHARED((N, 128), jnp.float32),
            *[pltpu.SemaphoreType.DMA for _ in range(2 * DEPTH)],
        ],
    )
    def scatter_kernel(emb_hbm, dst_hbm, x_hbm, out_hbm, *rest):
        idxs = rest[:DEPTH]
        rows = rest[DEPTH:2 * DEPTH]
        idx_t = rest[2 * DEPTH]
        rows_t = rest[2 * DEPTH + 1]
        acc = rest[2 * DEPTH + 2]
        isems = rest[2 * DEPTH + 3:2 * DEPTH + 3 + DEPTH]
        rsems = rest[2 * DEPTH + 3 + DEPTH:]
        c = lax.axis_index("c")
        s = lax.axis_index("s")
        row_lo = s * ROWS_T
        edge_base = s * PER_TILE

        def start_batch(b, j, col):
            lo = edge_base + b * BATCH
            pltpu.async_copy(dst_hbm.at[pl.ds(lo, BATCH)], idxs[j], isems[j])
            pltpu.async_copy(emb_hbm.at[pl.ds(lo, BATCH), pl.ds(col, 128)],
                             rows[j], rsems[j])

        for cl in range(4):
            col = (c * 4 + cl) * 128
            pltpu.sync_copy(x_hbm.at[pl.ds(row_lo, ROWS_T), pl.ds(col, 128)],
                            acc.at[pl.ds(row_lo, ROWS_T)])

            @pl.when(s == 15)
            def _():
                pltpu.sync_copy(
                    x_hbm.at[pl.ds(16 * ROWS_T, N - 16 * ROWS_T), pl.ds(col, 128)],
                    acc.at[pl.ds(16 * ROWS_T, N - 16 * ROWS_T)])

            plsc.subcore_barrier()

            for j in range(DEPTH):
                start_batch(j, j, col)

            def body(k, carry):
                for j in range(DEPTH):
                    b = k * DEPTH + j
                    pltpu.make_async_copy(dst_hbm.at[pl.ds(0, BATCH)],
                                          idxs[j], isems[j]).wait()
                    pltpu.make_async_copy(
                        emb_hbm.at[pl.ds(0, BATCH), pl.ds(col, 128)],
                        rows[j], rsems[j]).wait()
                    pltpu.sync_copy(rows[j], acc.at[idxs[j]], add=True)

                    @pl.when(b + DEPTH < NB_FULL)
                    def _():
                        start_batch(b + DEPTH, j, col)
                return carry

            lax.fori_loop(0, NB_FULL // DEPTH, body, 0)
            lo_t = edge_base + NB_FULL * BATCH
            pltpu.sync_copy(dst_hbm.at[pl.ds(lo_t, TAIL)], idx_t)
            pltpu.sync_copy(emb_hbm.at[pl.ds(lo_t, TAIL), pl.ds(col, 128)],
                            rows_t)
            pltpu.sync_copy(rows_t, acc.at[idx_t], add=True)

            plsc.subcore_barrier()
            pltpu.sync_copy(acc.at[pl.ds(row_lo, ROWS_T)],
                            out_hbm.at[pl.ds(row_lo, ROWS_T), pl.ds(col, 128)])

            @pl.when(s == 15)
            def _():
                pltpu.sync_copy(
                    acc.at[pl.ds(16 * ROWS_T, N - 16 * ROWS_T)],
                    out_hbm.at[pl.ds(16 * ROWS_T, N - 16 * ROWS_T), pl.ds(col, 128)])

            plsc.subcore_barrier()

    return scatter_kernel(emb2, dst, x2)


def kernel(x, x_edge, edge_distance, edge_index, wigner_inv,
           W1, b1, ln_g, ln_b, W2, b2, to_m, out_idx):
    emb = _compute_emb(x_edge, edge_distance, wigner_inv,
                       W1, b1, ln_g, ln_b, W2, b2)
    out2 = _sc_scatter(emb, edge_index[1], x.reshape(N, FULL * C))
    return out2.reshape(N, FULL, C)


# speedup vs baseline: 1.3973x; 1.3973x over previous
import functools

import numpy as np
import jax
import jax.numpy as jnp
from jax import lax
from jax.experimental import pallas as pl
from jax.experimental.pallas import tpu as pltpu
from jax.experimental.pallas import tpu_sc as plsc

N = 10000
E = 160000
FULL = 16
C = 64
CUTOFF = 12.0
RESCALE = 23.395238876342773
COLS = (0, 2, 6, 12)

PHASES = 2
EH = E // PHASES
B_EDGE = 1600
BATCH = 128
PER_TILE = EH // 16
NB_FULL = PER_TILE // BATCH
TAIL = PER_TILE - NB_FULL * BATCH
ROWS_T = 624
DEPTH = 2


def _emb_body(xe_ref, dist_ref, w4_ref, w1_ref, b1_ref, g_ref, bb_ref,
              w2_ref, b2_ref, p_ref, s_ref, out_ref):
    xe = xe_ref[...]
    bsz = xe.shape[0]
    h = jnp.dot(xe, w1_ref[...], preferred_element_type=jnp.float32) + b1_ref[...]
    mu = jnp.mean(h, axis=-1, keepdims=True)
    var = jnp.mean((h - mu) ** 2, axis=-1, keepdims=True)
    h = (h - mu) * lax.rsqrt(var + 1e-5) * g_ref[...] + bb_ref[...]
    h = h * (1.0 / (1.0 + jnp.exp(-h)))
    rad = jnp.dot(h, w2_ref[...], preferred_element_type=jnp.float32) + b2_ref[...]
    d = dist_ref[...].reshape(1, bsz) * (1.0 / CUTOFF)
    d5 = (d * d) * (d * d) * d
    env = 1.0 + d5 * (-21.0 + d * (35.0 - 15.0 * d))
    env = jnp.where(d < 1.0, env, 0.0) * (1.0 / RESCALE)
    rad = rad * jnp.transpose(env, (1, 0))
    w4 = jnp.dot(w4_ref[...], p_ref[...], preferred_element_type=jnp.float32)
    s_mat = s_ref[...]
    acc = None
    for l in range(4):
        wl = jnp.dot(w4[:, l * 16:(l + 1) * 16], s_mat,
                     preferred_element_type=jnp.float32)
        rl = rad[:, l * 64:(l + 1) * 64]
        r2 = jnp.tile(rl, (1, 16))
        term = wl * r2
        acc = term if acc is None else acc + term
    out_ref[...] = acc


def _compute_emb(phase, x_edge, dist3, wigf, W1, b1, ln_g, ln_b, W2, b2, p, s):
    grid = EH // B_EDGE
    off = phase * grid
    const = lambda shape: pl.BlockSpec(shape, lambda i: (0, 0))
    return pl.pallas_call(
        _emb_body,
        grid=(grid,),
        in_specs=[
            pl.BlockSpec((B_EDGE, 64), lambda i: (i + off, 0)),
            pl.BlockSpec((1, 1, B_EDGE), lambda i: (i + off, 0, 0)),
            pl.BlockSpec((B_EDGE, FULL * FULL), lambda i: (i + off, 0)),
            const((64, 64)),
            const((1, 64)),
            const((1, 64)),
            const((1, 64)),
            const((64, 256)),
            const((1, 256)),
            const((FULL * FULL, 64)),
            const((16, 1024)),
        ],
        out_specs=pl.BlockSpec((B_EDGE, 1024), lambda i: (i, 0)),
        out_shape=jax.ShapeDtypeStruct((EH, 1024), jnp.float32),
    )(x_edge, dist3, wigf, W1, b1.reshape(1, 64), ln_g.reshape(1, 64),
      ln_b.reshape(1, 64), W2, b2.reshape(1, 256), p, s)


def _sc_scatter(emb2, dst, base2):
    mesh = plsc.VectorSubcoreMesh(core_axis_name="c", subcore_axis_name="s")

    @functools.partial(
        pl.kernel,
        mesh=mesh,
        out_type=jax.ShapeDtypeStruct((N, 1024), jnp.float32),
        scratch_types=[
            *[pltpu.VMEM((BATCH,), jnp.int32) for _ in range(DEPTH)],
            *[pltpu.VMEM((BATCH, 128), jnp.float32) for _ in range(DEPTH)],
            pltpu.VMEM((TAIL,), jnp.int32),
            pltpu.VMEM((TAIL, 128), jnp.float32),
            pltpu.VMEM_SHARED((N, 128), jnp.float32),
            *[pltpu.SemaphoreType.DMA for _ in range(2 * DEPTH)],
        ],
    )
    def scatter_kernel(emb_hbm, dst_hbm, x_hbm, out_hbm, *rest):
        idxs = rest[:DEPTH]
        rows = rest[DEPTH:2 * DEPTH]
        idx_t = rest[2 * DEPTH]
        rows_t = rest[2 * DEPTH + 1]
        acc = rest[2 * DEPTH + 2]
        isems = rest[2 * DEPTH + 3:2 * DEPTH + 3 + DEPTH]
        rsems = rest[2 * DEPTH + 3 + DEPTH:]
        c = lax.axis_index("c")
        s = lax.axis_index("s")
        row_lo = s * ROWS_T
        edge_base = s * PER_TILE

        def start_batch(b, j, col):
            lo = edge_base + b * BATCH
            pltpu.async_copy(dst_hbm.at[pl.ds(lo, BATCH)], idxs[j], isems[j])
            pltpu.async_copy(emb_hbm.at[pl.ds(lo, BATCH), pl.ds(col, 128)],
                             rows[j], rsems[j])

        def finish_batch(j, col):
            pltpu.make_async_copy(dst_hbm.at[pl.ds(0, BATCH)],
                                  idxs[j], isems[j]).wait()
            pltpu.make_async_copy(emb_hbm.at[pl.ds(0, BATCH), pl.ds(col, 128)],
                                  rows[j], rsems[j]).wait()
            pltpu.sync_copy(rows[j], acc.at[idxs[j]], add=True)

        for cl in range(4):
            col = (c * 4 + cl) * 128
            pltpu.sync_copy(x_hbm.at[pl.ds(row_lo, ROWS_T), pl.ds(col, 128)],
                            acc.at[pl.ds(row_lo, ROWS_T)])

            @pl.when(s == 15)
            def _():
                pltpu.sync_copy(
                    x_hbm.at[pl.ds(16 * ROWS_T, N - 16 * ROWS_T), pl.ds(col, 128)],
                    acc.at[pl.ds(16 * ROWS_T, N - 16 * ROWS_T)])

            plsc.subcore_barrier()

            for j in range(DEPTH):
                start_batch(j, j, col)

            def body(k, carry):
                for j in range(DEPTH):
                    b = k * DEPTH + j
                    finish_batch(j, col)

                    @pl.when(b + DEPTH < NB_FULL)
                    def _():
                        start_batch(b + DEPTH, j, col)
                return carry

            lax.fori_loop(0, NB_FULL // DEPTH, body, 0)
            for b in range((NB_FULL // DEPTH) * DEPTH, NB_FULL):
                finish_batch(b % DEPTH, col)
            lo_t = edge_base + NB_FULL * BATCH
            pltpu.sync_copy(dst_hbm.at[pl.ds(lo_t, TAIL)], idx_t)
            pltpu.sync_copy(emb_hbm.at[pl.ds(lo_t, TAIL), pl.ds(col, 128)],
                            rows_t)
            pltpu.sync_copy(rows_t, acc.at[idx_t], add=True)

            plsc.subcore_barrier()
            pltpu.sync_copy(acc.at[pl.ds(row_lo, ROWS_T)],
                            out_hbm.at[pl.ds(row_lo, ROWS_T), pl.ds(col, 128)])

            @pl.when(s == 15)
            def _():
                pltpu.sync_copy(
                    acc.at[pl.ds(16 * ROWS_T, N - 16 * ROWS_T)],
                    out_hbm.at[pl.ds(16 * ROWS_T, N - 16 * ROWS_T), pl.ds(col, 128)])

            plsc.subcore_barrier()

    return scatter_kernel(emb2, dst, base2)


def kernel(x, x_edge, edge_distance, edge_index, wigner_inv,
           W1, b1, ln_g, ln_b, W2, b2, to_m, out_idx):
    wigf = wigner_inv.reshape(E, FULL * FULL)
    dist3 = edge_distance.reshape(E // B_EDGE, 1, B_EDGE)
    p = np.zeros((FULL * FULL, 64), dtype=np.float32)
    for l, col in enumerate(COLS):
        for i in range(FULL):
            p[i * FULL + col, l * 16 + i] = 1.0
    p = jnp.asarray(p)
    s = np.zeros((16, 1024), dtype=np.float32)
    for f in range(1024):
        s[f // 64, f] = 1.0
    s = jnp.asarray(s)
    dst = edge_index[1]
    base = x.reshape(N, FULL * C)
    for ph in range(PHASES):
        emb = _compute_emb(ph, x_edge, dist3, wigf, W1, b1, ln_g, ln_b, W2, b2,
                           p, s)
        base = _sc_scatter(emb, lax.dynamic_slice_in_dim(dst, ph * EH, EH), base)
    return base.reshape(N, FULL, C)


# speedup vs baseline: 1.4148x; 1.0125x over previous
import functools

import numpy as np
import jax
import jax.numpy as jnp
from jax import lax
from jax.experimental import pallas as pl
from jax.experimental.pallas import tpu as pltpu
from jax.experimental.pallas import tpu_sc as plsc

N = 10000
E = 160000
FULL = 16
C = 64
CUTOFF = 12.0
RESCALE = 23.395238876342773
COLS = (0, 2, 6, 12)

PHASE_E = (44800, 54400, 60800)
B_EDGE = 1600
BATCH = 128
ROWS_T = 624
DEPTH = 2


def _emb_body(xe_ref, dist_ref, w4_ref, w1_ref, b1_ref, g_ref, bb_ref,
              w2_ref, b2_ref, p_ref, s_ref, out_ref):
    xe = xe_ref[...]
    bsz = xe.shape[0]
    h = jnp.dot(xe, w1_ref[...], preferred_element_type=jnp.float32) + b1_ref[...]
    mu = jnp.mean(h, axis=-1, keepdims=True)
    var = jnp.mean((h - mu) ** 2, axis=-1, keepdims=True)
    h = (h - mu) * lax.rsqrt(var + 1e-5) * g_ref[...] + bb_ref[...]
    h = h * (1.0 / (1.0 + jnp.exp(-h)))
    rad = jnp.dot(h, w2_ref[...], preferred_element_type=jnp.float32) + b2_ref[...]
    d = dist_ref[...].reshape(1, bsz) * (1.0 / CUTOFF)
    d5 = (d * d) * (d * d) * d
    env = 1.0 + d5 * (-21.0 + d * (35.0 - 15.0 * d))
    env = jnp.where(d < 1.0, env, 0.0) * (1.0 / RESCALE)
    rad = rad * jnp.transpose(env, (1, 0))
    w4 = jnp.dot(w4_ref[...], p_ref[...], preferred_element_type=jnp.float32)
    s_mat = s_ref[...]
    acc = None
    for l in range(4):
        wl = jnp.dot(w4[:, l * 16:(l + 1) * 16], s_mat,
                     preferred_element_type=jnp.float32)
        rl = rad[:, l * 64:(l + 1) * 64]
        r2 = jnp.tile(rl, (1, 16))
        term = wl * r2
        acc = term if acc is None else acc + term
    out_ref[...] = acc


def _compute_emb(off, grid, x_edge, dist3, wigf, W1, b1, ln_g, ln_b, W2, b2, p, s):
    const = lambda shape: pl.BlockSpec(shape, lambda i: (0, 0))
    return pl.pallas_call(
        _emb_body,
        grid=(grid,),
        in_specs=[
            pl.BlockSpec((B_EDGE, 64), lambda i: (i + off, 0)),
            pl.BlockSpec((1, 1, B_EDGE), lambda i: (i + off, 0, 0)),
            pl.BlockSpec((B_EDGE, FULL * FULL), lambda i: (i + off, 0)),
            const((64, 64)),
            const((1, 64)),
            const((1, 64)),
            const((1, 64)),
            const((64, 256)),
            const((1, 256)),
            const((FULL * FULL, 64)),
            const((16, 1024)),
        ],
        out_specs=pl.BlockSpec((B_EDGE, 1024), lambda i: (i, 0)),
        out_shape=jax.ShapeDtypeStruct((grid * B_EDGE, 1024), jnp.float32),
    )(x_edge, dist3, wigf, W1, b1.reshape(1, 64), ln_g.reshape(1, 64),
      ln_b.reshape(1, 64), W2, b2.reshape(1, 256), p, s)


def _sc_scatter(emb2, dst, base2, eh):
    per_tile = eh // 16
    nb_full = per_tile // BATCH
    tail = per_tile - nb_full * BATCH
    mesh = plsc.VectorSubcoreMesh(core_axis_name="c", subcore_axis_name="s")

    @functools.partial(
        pl.kernel,
        mesh=mesh,
        out_type=jax.ShapeDtypeStruct((N, 1024), jnp.float32),
        scratch_types=[
            *[pltpu.VMEM((BATCH,), jnp.int32) for _ in range(DEPTH)],
            *[pltpu.VMEM((BATCH, 128), jnp.float32) for _ in range(DEPTH)],
            pltpu.VMEM((tail,), jnp.int32),
            pltpu.VMEM((tail, 128), jnp.float32),
            pltpu.VMEM_SHARED((N, 128), jnp.float32),
            *[pltpu.SemaphoreType.DMA for _ in range(2 * DEPTH)],
        ],
    )
    def scatter_kernel(emb_hbm, dst_hbm, x_hbm, out_hbm, *rest):
        idxs = rest[:DEPTH]
        rows = rest[DEPTH:2 * DEPTH]
        idx_t = rest[2 * DEPTH]
        rows_t = rest[2 * DEPTH + 1]
        acc = rest[2 * DEPTH + 2]
        isems = rest[2 * DEPTH + 3:2 * DEPTH + 3 + DEPTH]
        rsems = rest[2 * DEPTH + 3 + DEPTH:]
        c = lax.axis_index("c")
        s = lax.axis_index("s")
        row_lo = s * ROWS_T
        edge_base = s * per_tile

        def start_batch(b, j, col):
            lo = edge_base + b * BATCH
            pltpu.async_copy(dst_hbm.at[pl.ds(lo, BATCH)], idxs[j], isems[j])
            pltpu.async_copy(emb_hbm.at[pl.ds(lo, BATCH), pl.ds(col, 128)],
                             rows[j], rsems[j])

        def finish_batch(j, col):
            pltpu.make_async_copy(dst_hbm.at[pl.ds(0, BATCH)],
                                  idxs[j], isems[j]).wait()
            pltpu.make_async_copy(emb_hbm.at[pl.ds(0, BATCH), pl.ds(col, 128)],
                                  rows[j], rsems[j]).wait()
            pltpu.sync_copy(rows[j], acc.at[idxs[j]], add=True)

        for cl in range(4):
            col = (c * 4 + cl) * 128
            pltpu.sync_copy(x_hbm.at[pl.ds(row_lo, ROWS_T), pl.ds(col, 128)],
                            acc.at[pl.ds(row_lo, ROWS_T)])

            @pl.when(s == 15)
            def _():
                pltpu.sync_copy(
                    x_hbm.at[pl.ds(16 * ROWS_T, N - 16 * ROWS_T), pl.ds(col, 128)],
                    acc.at[pl.ds(16 * ROWS_T, N - 16 * ROWS_T)])

            plsc.subcore_barrier()

            for j in range(DEPTH):
                start_batch(j, j, col)

            def body(k, carry):
                for j in range(DEPTH):
                    b = k * DEPTH + j
                    finish_batch(j, col)

                    @pl.when(b + DEPTH < nb_full)
                    def _():
                        start_batch(b + DEPTH, j, col)
                return carry

            lax.fori_loop(0, nb_full // DEPTH, body, 0)
            for b in range((nb_full // DEPTH) * DEPTH, nb_full):
                finish_batch(b % DEPTH, col)
            lo_t = edge_base + nb_full * BATCH
            pltpu.sync_copy(dst_hbm.at[pl.ds(lo_t, tail)], idx_t)
            pltpu.sync_copy(emb_hbm.at[pl.ds(lo_t, tail), pl.ds(col, 128)],
                            rows_t)
            pltpu.sync_copy(rows_t, acc.at[idx_t], add=True)

            plsc.subcore_barrier()
            pltpu.sync_copy(acc.at[pl.ds(row_lo, ROWS_T)],
                            out_hbm.at[pl.ds(row_lo, ROWS_T), pl.ds(col, 128)])

            @pl.when(s == 15)
            def _():
                pltpu.sync_copy(
                    acc.at[pl.ds(16 * ROWS_T, N - 16 * ROWS_T)],
                    out_hbm.at[pl.ds(16 * ROWS_T, N - 16 * ROWS_T), pl.ds(col, 128)])

            plsc.subcore_barrier()

    return scatter_kernel(emb2, dst, base2)


def kernel(x, x_edge, edge_distance, edge_index, wigner_inv,
           W1, b1, ln_g, ln_b, W2, b2, to_m, out_idx):
    wigf = wigner_inv.reshape(E, FULL * FULL)
    dist3 = edge_distance.reshape(E // B_EDGE, 1, B_EDGE)
    p = np.zeros((FULL * FULL, 64), dtype=np.float32)
    for l, col in enumerate(COLS):
        for i in range(FULL):
            p[i * FULL + col, l * 16 + i] = 1.0
    p = jnp.asarray(p)
    s = np.zeros((16, 1024), dtype=np.float32)
    for f in range(1024):
        s[f // 64, f] = 1.0
    s = jnp.asarray(s)
    dst = edge_index[1]
    base = x.reshape(N, FULL * C)
    e_off = 0
    for eh in PHASE_E:
        emb = _compute_emb(e_off // B_EDGE, eh // B_EDGE, x_edge, dist3, wigf,
                           W1, b1, ln_g, ln_b, W2, b2, p, s)
        base = _sc_scatter(emb, lax.dynamic_slice_in_dim(dst, e_off, eh),
                           base, eh)
        e_off += eh
    return base.reshape(N, FULL, C)


# speedup vs baseline: 1.4365x; 1.0154x over previous
import functools

import numpy as np
import jax
import jax.numpy as jnp
from jax import lax
from jax.experimental import pallas as pl
from jax.experimental.pallas import tpu as pltpu
from jax.experimental.pallas import tpu_sc as plsc

N = 10000
E = 160000
FULL = 16
C = 64
CUTOFF = 12.0
RESCALE = 23.395238876342773
COLS = (0, 2, 6, 12)

PHASE_E = (44800, 54400, 60800)
B_EDGE = 1600
BATCH = 128
ROWS_T = 624
DEPTH = 2


def _emb_body(xe_ref, dist_ref, w4_ref, w1_ref, b1_ref, g_ref, bb_ref,
              w2_ref, b2_ref, p_ref, s_ref, out_ref):
    xe = xe_ref[...]
    bsz = xe.shape[0]
    h = jnp.dot(xe, w1_ref[...], preferred_element_type=jnp.float32) + b1_ref[...]
    mu = jnp.mean(h, axis=-1, keepdims=True)
    var = jnp.mean((h - mu) ** 2, axis=-1, keepdims=True)
    h = (h - mu) * lax.rsqrt(var + 1e-5) * g_ref[...] + bb_ref[...]
    h = h * (1.0 / (1.0 + jnp.exp(-h)))
    rad = jnp.dot(h, w2_ref[...], preferred_element_type=jnp.float32) + b2_ref[...]
    d = dist_ref[...].reshape(1, bsz) * (1.0 / CUTOFF)
    d5 = (d * d) * (d * d) * d
    env = 1.0 + d5 * (-21.0 + d * (35.0 - 15.0 * d))
    env = jnp.where(d < 1.0, env, 0.0) * (1.0 / RESCALE)
    rad = rad * jnp.transpose(env, (1, 0))
    w4 = jnp.dot(w4_ref[...], p_ref[...], preferred_element_type=jnp.float32)
    s_mat = s_ref[...]
    acc = None
    for l in range(4):
        wl = jnp.dot(w4[:, l * 16:(l + 1) * 16], s_mat,
                     preferred_element_type=jnp.float32)
        rl = rad[:, l * 64:(l + 1) * 64]
        r2 = jnp.tile(rl, (1, 16))
        term = wl * r2
        acc = term if acc is None else acc + term
    out_ref[...] = acc


def _compute_emb(off, grid, x_edge, dist3, wigf, W1, b1, ln_g, ln_b, W2, b2, p, s):
    const = lambda shape: pl.BlockSpec(shape, lambda i: (0, 0))
    return pl.pallas_call(
        _emb_body,
        grid=(grid,),
        in_specs=[
            pl.BlockSpec((B_EDGE, 64), lambda i: (i + off, 0)),
            pl.BlockSpec((1, 1, B_EDGE), lambda i: (i + off, 0, 0)),
            pl.BlockSpec((B_EDGE, FULL * FULL), lambda i: (i + off, 0)),
            const((64, 64)),
            const((1, 64)),
            const((1, 64)),
            const((1, 64)),
            const((64, 256)),
            const((1, 256)),
            const((FULL * FULL, 64)),
            const((16, 1024)),
        ],
        out_specs=pl.BlockSpec((B_EDGE, 1024), lambda i: (i, 0)),
        out_shape=jax.ShapeDtypeStruct((grid * B_EDGE, 1024), jnp.float32),
    )(x_edge, dist3, wigf, W1, b1.reshape(1, 64), ln_g.reshape(1, 64),
      ln_b.reshape(1, 64), W2, b2.reshape(1, 256), p, s)


def _sc_scatter(emb2, dst, base2, eh):
    per_tile = eh // 16
    nb_full = per_tile // BATCH
    tail = per_tile - nb_full * BATCH
    mesh = plsc.VectorSubcoreMesh(core_axis_name="c", subcore_axis_name="s")

    @functools.partial(
        pl.kernel,
        mesh=mesh,
        out_type=jax.ShapeDtypeStruct((N, 1024), jnp.float32),
        scratch_types=[
            *[pltpu.VMEM((BATCH,), jnp.int32) for _ in range(DEPTH)],
            *[pltpu.VMEM((BATCH, 128), jnp.float32) for _ in range(DEPTH)],
            pltpu.VMEM((tail,), jnp.int32),
            pltpu.VMEM((tail, 128), jnp.float32),
            pltpu.VMEM_SHARED((N, 128), jnp.float32),
            *[pltpu.SemaphoreType.DMA for _ in range(2 * DEPTH + 2)],
        ],
    )
    def scatter_kernel(emb_hbm, dst_hbm, x_hbm, out_hbm, *rest):
        idxs = rest[:DEPTH]
        rows = rest[DEPTH:2 * DEPTH]
        idx_t = rest[2 * DEPTH]
        rows_t = rest[2 * DEPTH + 1]
        acc = rest[2 * DEPTH + 2]
        isems = rest[2 * DEPTH + 3:2 * DEPTH + 3 + DEPTH]
        rsems = rest[2 * DEPTH + 3 + DEPTH:2 * DEPTH + 3 + 2 * DEPTH]
        tsem_i, tsem_r = rest[2 * DEPTH + 3 + 2 * DEPTH:]
        c = lax.axis_index("c")
        s = lax.axis_index("s")
        row_lo = s * ROWS_T
        edge_base = s * per_tile

        def start_batch(b, j, col):
            lo = edge_base + b * BATCH
            pltpu.async_copy(dst_hbm.at[pl.ds(lo, BATCH)], idxs[j], isems[j])
            pltpu.async_copy(emb_hbm.at[pl.ds(lo, BATCH), pl.ds(col, 128)],
                             rows[j], rsems[j])

        def finish_batch(j, col):
            pltpu.make_async_copy(dst_hbm.at[pl.ds(0, BATCH)],
                                  idxs[j], isems[j]).wait()
            pltpu.make_async_copy(emb_hbm.at[pl.ds(0, BATCH), pl.ds(col, 128)],
                                  rows[j], rsems[j]).wait()
            pltpu.sync_copy(rows[j], acc.at[idxs[j]], add=True)

        for cl in range(4):
            col = (c * 4 + cl) * 128
            pltpu.sync_copy(x_hbm.at[pl.ds(row_lo, ROWS_T), pl.ds(col, 128)],
                            acc.at[pl.ds(row_lo, ROWS_T)])

            @pl.when(s == 15)
            def _():
                pltpu.sync_copy(
                    x_hbm.at[pl.ds(16 * ROWS_T, N - 16 * ROWS_T), pl.ds(col, 128)],
                    acc.at[pl.ds(16 * ROWS_T, N - 16 * ROWS_T)])

            plsc.subcore_barrier()

            for j in range(DEPTH):
                start_batch(j, j, col)
            lo_t = edge_base + nb_full * BATCH
            pltpu.async_copy(dst_hbm.at[pl.ds(lo_t, tail)], idx_t, tsem_i)
            pltpu.async_copy(emb_hbm.at[pl.ds(lo_t, tail), pl.ds(col, 128)],
                             rows_t, tsem_r)

            def body(k, carry):
                for j in range(DEPTH):
                    b = k * DEPTH + j
                    finish_batch(j, col)

                    @pl.when(b + DEPTH < nb_full)
                    def _():
                        start_batch(b + DEPTH, j, col)
                return carry

            lax.fori_loop(0, nb_full // DEPTH, body, 0)
            for b in range((nb_full // DEPTH) * DEPTH, nb_full):
                finish_batch(b % DEPTH, col)
            pltpu.make_async_copy(dst_hbm.at[pl.ds(0, tail)],
                                  idx_t, tsem_i).wait()
            pltpu.make_async_copy(emb_hbm.at[pl.ds(0, tail), pl.ds(col, 128)],
                                  rows_t, tsem_r).wait()
            pltpu.sync_copy(rows_t, acc.at[idx_t], add=True)

            plsc.subcore_barrier()
            pltpu.sync_copy(acc.at[pl.ds(row_lo, ROWS_T)],
                            out_hbm.at[pl.ds(row_lo, ROWS_T), pl.ds(col, 128)])

            @pl.when(s == 15)
            def _():
                pltpu.sync_copy(
                    acc.at[pl.ds(16 * ROWS_T, N - 16 * ROWS_T)],
                    out_hbm.at[pl.ds(16 * ROWS_T, N - 16 * ROWS_T), pl.ds(col, 128)])

            plsc.subcore_barrier()

    return scatter_kernel(emb2, dst, base2)


def kernel(x, x_edge, edge_distance, edge_index, wigner_inv,
           W1, b1, ln_g, ln_b, W2, b2, to_m, out_idx):
    wigf = wigner_inv.reshape(E, FULL * FULL)
    dist3 = edge_distance.reshape(E // B_EDGE, 1, B_EDGE)
    p = np.zeros((FULL * FULL, 64), dtype=np.float32)
    for l, col in enumerate(COLS):
        for i in range(FULL):
            p[i * FULL + col, l * 16 + i] = 1.0
    p = jnp.asarray(p)
    s = np.zeros((16, 1024), dtype=np.float32)
    for f in range(1024):
        s[f // 64, f] = 1.0
    s = jnp.asarray(s)
    dst = edge_index[1]
    base = x.reshape(N, FULL * C)
    e_off = 0
    for eh in PHASE_E:
        emb = _compute_emb(e_off // B_EDGE, eh // B_EDGE, x_edge, dist3, wigf,
                           W1, b1, ln_g, ln_b, W2, b2, p, s)
        base = _sc_scatter(emb, lax.dynamic_slice_in_dim(dst, e_off, eh),
                           base, eh)
        e_off += eh
    return base.reshape(N, FULL, C)


# speedup vs baseline: 1.4461x; 1.0067x over previous
import functools

import numpy as np
import jax
import jax.numpy as jnp
from jax import lax
from jax.experimental import pallas as pl
from jax.experimental.pallas import tpu as pltpu
from jax.experimental.pallas import tpu_sc as plsc

N = 10000
E = 160000
FULL = 16
C = 64
CUTOFF = 12.0
RESCALE = 23.395238876342773
COLS = (0, 2, 6, 12)

PHASE_E = (32000, 57600, 70400)
B_EDGE = 1600
BATCH = 128
ROWS_T = 624
DEPTH = 2


def _emb_body(xe_ref, dist_ref, w4_ref, w1_ref, b1_ref, g_ref, bb_ref,
              w2_ref, b2_ref, p_ref, s_ref, out_ref):
    xe = xe_ref[...]
    bsz = xe.shape[0]
    h = jnp.dot(xe, w1_ref[...], preferred_element_type=jnp.float32) + b1_ref[...]
    mu = jnp.mean(h, axis=-1, keepdims=True)
    var = jnp.mean((h - mu) ** 2, axis=-1, keepdims=True)
    h = (h - mu) * lax.rsqrt(var + 1e-5) * g_ref[...] + bb_ref[...]
    h = h * (1.0 / (1.0 + jnp.exp(-h)))
    rad = jnp.dot(h, w2_ref[...], preferred_element_type=jnp.float32) + b2_ref[...]
    d = dist_ref[...].reshape(1, bsz) * (1.0 / CUTOFF)
    d5 = (d * d) * (d * d) * d
    env = 1.0 + d5 * (-21.0 + d * (35.0 - 15.0 * d))
    env = jnp.where(d < 1.0, env, 0.0) * (1.0 / RESCALE)
    rad = rad * jnp.transpose(env, (1, 0))
    w4 = jnp.dot(w4_ref[...], p_ref[...], preferred_element_type=jnp.float32)
    s_mat = s_ref[...]
    acc = None
    for l in range(4):
        wl = jnp.dot(w4[:, l * 16:(l + 1) * 16], s_mat,
                     preferred_element_type=jnp.float32)
        rl = rad[:, l * 64:(l + 1) * 64]
        r2 = jnp.tile(rl, (1, 16))
        term = wl * r2
        acc = term if acc is None else acc + term
    out_ref[...] = acc


def _compute_emb(off, grid, x_edge, dist3, wigf, W1, b1, ln_g, ln_b, W2, b2, p, s):
    const = lambda shape: pl.BlockSpec(shape, lambda i: (0, 0))
    return pl.pallas_call(
        _emb_body,
        grid=(grid,),
        in_specs=[
            pl.BlockSpec((B_EDGE, 64), lambda i: (i + off, 0)),
            pl.BlockSpec((1, 1, B_EDGE), lambda i: (i + off, 0, 0)),
            pl.BlockSpec((B_EDGE, FULL * FULL), lambda i: (i + off, 0)),
            const((64, 64)),
            const((1, 64)),
            const((1, 64)),
            const((1, 64)),
            const((64, 256)),
            const((1, 256)),
            const((FULL * FULL, 64)),
            const((16, 1024)),
        ],
        out_specs=pl.BlockSpec((B_EDGE, 1024), lambda i: (i, 0)),
        out_shape=jax.ShapeDtypeStruct((grid * B_EDGE, 1024), jnp.float32),
    )(x_edge, dist3, wigf, W1, b1.reshape(1, 64), ln_g.reshape(1, 64),
      ln_b.reshape(1, 64), W2, b2.reshape(1, 256), p, s)


def _sc_scatter(emb2, dst, base2, eh):
    per_tile = eh // 16
    nb_full = per_tile // BATCH
    tail = per_tile - nb_full * BATCH
    mesh = plsc.VectorSubcoreMesh(core_axis_name="c", subcore_axis_name="s")

    @functools.partial(
        pl.kernel,
        mesh=mesh,
        out_type=jax.ShapeDtypeStruct((N, 1024), jnp.float32),
        scratch_types=[
            *[pltpu.VMEM((BATCH,), jnp.int32) for _ in range(DEPTH)],
            *[pltpu.VMEM((BATCH, 128), jnp.float32) for _ in range(DEPTH)],
            pltpu.VMEM((tail,), jnp.int32),
            pltpu.VMEM((tail, 128), jnp.float32),
            pltpu.VMEM_SHARED((N, 128), jnp.float32),
            *[pltpu.SemaphoreType.DMA for _ in range(2 * DEPTH + 2)],
        ],
    )
    def scatter_kernel(emb_hbm, dst_hbm, x_hbm, out_hbm, *rest):
        idxs = rest[:DEPTH]
        rows = rest[DEPTH:2 * DEPTH]
        idx_t = rest[2 * DEPTH]
        rows_t = rest[2 * DEPTH + 1]
        acc = rest[2 * DEPTH + 2]
        isems = rest[2 * DEPTH + 3:2 * DEPTH + 3 + DEPTH]
        rsems = rest[2 * DEPTH + 3 + DEPTH:2 * DEPTH + 3 + 2 * DEPTH]
        tsem_i, tsem_r = rest[2 * DEPTH + 3 + 2 * DEPTH:]
        c = lax.axis_index("c")
        s = lax.axis_index("s")
        row_lo = s * ROWS_T
        edge_base = s * per_tile

        def start_batch(b, j, col):
            lo = edge_base + b * BATCH
            pltpu.async_copy(dst_hbm.at[pl.ds(lo, BATCH)], idxs[j], isems[j])
            pltpu.async_copy(emb_hbm.at[pl.ds(lo, BATCH), pl.ds(col, 128)],
                             rows[j], rsems[j])

        def finish_batch(j, col):
            pltpu.make_async_copy(dst_hbm.at[pl.ds(0, BATCH)],
                                  idxs[j], isems[j]).wait()
            pltpu.make_async_copy(emb_hbm.at[pl.ds(0, BATCH), pl.ds(col, 128)],
                                  rows[j], rsems[j]).wait()
            pltpu.sync_copy(rows[j], acc.at[idxs[j]], add=True)

        for cl in range(4):
            col = (c * 4 + cl) * 128
            pltpu.sync_copy(x_hbm.at[pl.ds(row_lo, ROWS_T), pl.ds(col, 128)],
                            acc.at[pl.ds(row_lo, ROWS_T)])

            @pl.when(s == 15)
            def _():
                pltpu.sync_copy(
                    x_hbm.at[pl.ds(16 * ROWS_T, N - 16 * ROWS_T), pl.ds(col, 128)],
                    acc.at[pl.ds(16 * ROWS_T, N - 16 * ROWS_T)])

            plsc.subcore_barrier()

            for j in range(DEPTH):
                start_batch(j, j, col)
            lo_t = edge_base + nb_full * BATCH
            pltpu.async_copy(dst_hbm.at[pl.ds(lo_t, tail)], idx_t, tsem_i)
            pltpu.async_copy(emb_hbm.at[pl.ds(lo_t, tail), pl.ds(col, 128)],
                             rows_t, tsem_r)

            def body(k, carry):
                for j in range(DEPTH):
                    b = k * DEPTH + j
                    finish_batch(j, col)

                    @pl.when(b + DEPTH < nb_full)
                    def _():
                        start_batch(b + DEPTH, j, col)
                return carry

            lax.fori_loop(0, nb_full // DEPTH, body, 0)
            for b in range((nb_full // DEPTH) * DEPTH, nb_full):
                finish_batch(b % DEPTH, col)
            pltpu.make_async_copy(dst_hbm.at[pl.ds(0, tail)],
                                  idx_t, tsem_i).wait()
            pltpu.make_async_copy(emb_hbm.at[pl.ds(0, tail), pl.ds(col, 128)],
                                  rows_t, tsem_r).wait()
            pltpu.sync_copy(rows_t, acc.at[idx_t], add=True)

            plsc.subcore_barrier()
            pltpu.sync_copy(acc.at[pl.ds(row_lo, ROWS_T)],
                            out_hbm.at[pl.ds(row_lo, ROWS_T), pl.ds(col, 128)])

            @pl.when(s == 15)
            def _():
                pltpu.sync_copy(
                    acc.at[pl.ds(16 * ROWS_T, N - 16 * ROWS_T)],
                    out_hbm.at[pl.ds(16 * ROWS_T, N - 16 * ROWS_T), pl.ds(col, 128)])

            plsc.subcore_barrier()

    return scatter_kernel(emb2, dst, base2)


def kernel(x, x_edge, edge_distance, edge_index, wigner_inv,
           W1, b1, ln_g, ln_b, W2, b2, to_m, out_idx):
    wigf = wigner_inv.reshape(E, FULL * FULL)
    dist3 = edge_distance.reshape(E // B_EDGE, 1, B_EDGE)
    p = np.zeros((FULL * FULL, 64), dtype=np.float32)
    for l, col in enumerate(COLS):
        for i in range(FULL):
            p[i * FULL + col, l * 16 + i] = 1.0
    p = jnp.asarray(p)
    s = np.zeros((16, 1024), dtype=np.float32)
    for f in range(1024):
        s[f // 64, f] = 1.0
    s = jnp.asarray(s)
    dst = edge_index[1]
    base = x.reshape(N, FULL * C)
    e_off = 0
    for eh in PHASE_E:
        emb = _compute_emb(e_off // B_EDGE, eh // B_EDGE, x_edge, dist3, wigf,
                           W1, b1, ln_g, ln_b, W2, b2, p, s)
        base = _sc_scatter(emb, lax.dynamic_slice_in_dim(dst, e_off, eh),
                           base, eh)
        e_off += eh
    return base.reshape(N, FULL, C)
